# baseline (speedup 1.0000x reference)
import jax
import jax.numpy as jnp
import numpy as np
from jax.experimental import pallas as pl
from jax.experimental.pallas import tpu as pltpu

EPS_LN = 1e-5
EPS_SUMNORM = 1e-5
B, T, F, H = 8, 1024, 128, 64
C = 128


def _fwp_kernel(x_ref, state_ref, wk_ref, wq_ref, wv_ref,
                g_ref, beta_ref, w1_ref, b1_ref, w2_ref, b2_ref, ws_ref,
                bs_ref, lexp_ref, l32_ref, y_ref, s_ref, cm_ref):
    tc = pl.program_id(1)

    @pl.when(tc == 0)
    def _init():
        cm_ref[...] = state_ref[0]

    x = x_ref[0]
    mu = jnp.mean(x, axis=1, keepdims=True)
    xc = x - mu
    var = jnp.mean(xc * xc, axis=1, keepdims=True)
    xn = xc * jax.lax.rsqrt(var + EPS_LN) * g_ref[...] + beta_ref[...]

    dg_t = lambda a, w: jax.lax.dot_general(
        a, w, (((1,), (1,)), ((), ())), preferred_element_type=jnp.float32)
    K = jnp.maximum(dg_t(xn, wk_ref[...]), 0.0)
    Q = jnp.maximum(dg_t(xn, wq_ref[...]), 0.0)
    V = dg_t(xn, wv_ref[...])
    K = K / (EPS_SUMNORM + jnp.sum(K, axis=1, keepdims=True))
    Q = Q / (EPS_SUMNORM + jnp.sum(Q, axis=1, keepdims=True))

    cm = cm_ref[...]

    vt = V.T.astype(jnp.bfloat16)
    vt2 = jnp.concatenate(
        [vt, jnp.ones((H, 2 * H), jnp.bfloat16)], axis=1)
    vtile = jnp.tile(vt2, (C, 1))
    m = lexp_ref[...] * vtile

    hi = cm.astype(jnp.bfloat16)
    lo = (cm - hi.astype(jnp.float32)).astype(jnp.bfloat16)
    krhs = jnp.concatenate(
        [K.astype(jnp.bfloat16), hi, lo], axis=0)

    s2 = jax.lax.dot_general(
        m, krhs, (((1,), (0,)), ((), ())),
        preferred_element_type=jnp.float32)
    s_ref[0] = s2.reshape(C, H, H)

    a = dg_t(Q, K) * l32_ref[...]
    y_intra = jax.lax.dot_general(
        a, V, (((1,), (0,)), ((), ())), preferred_element_type=jnp.float32)
    y = dg_t(Q, cm) + y_intra

    h = jnp.maximum(dg_t(y, w1_ref[...]) + b1_ref[...], 0.0)
    h = jnp.maximum(dg_t(h, w2_ref[...]) + b2_ref[...], 0.0)
    y_ref[0] = h + dg_t(xn, ws_ref[...]) + bs_ref[...]

    cm_ref[...] = cm + jax.lax.dot_general(
        V, K, (((0,), (0,)), ((), ())), preferred_element_type=jnp.float32)


@jax.jit
def kernel(x, state, Wk, Wq, Wv, gamma, beta, W1, b1, W2, b2, Ws, bs):
    r = np.arange(C * H)
    t_idx, i_idx = r // H, r % H
    s_idx = np.arange(C + 2 * H)
    lexp = np.zeros((C * H, C + 2 * H), np.float32)
    lexp[:, :C] = (s_idx[None, :C] <= t_idx[:, None]).astype(np.float32)
    lexp[:, C:C + H] = (s_idx[None, C:C + H] - C == i_idx[:, None])
    lexp[:, C + H:] = (s_idx[None, C + H:] - C - H == i_idx[:, None])
    lexp = jnp.asarray(lexp, dtype=jnp.bfloat16)
    l32 = jnp.asarray(np.tril(np.ones((C, C), np.float32)))

    state_m = state.reshape(B, H, H)

    full = lambda shp: pl.BlockSpec(shp, lambda b, t: (0,) * len(shp))
    in_specs = [
        pl.BlockSpec((1, C, F), lambda b, t: (b, t, 0)),
        pl.BlockSpec((1, H, H), lambda b, t: (b, 0, 0)),
        full((H, F)), full((H, F)), full((H, F)),
        full((1, F)), full((1, F)),
        full((H, H)), full((1, H)),
        full((H, H)), full((1, H)),
        full((H, F)), full((1, H)),
        full((C * H, C + 2 * H)),
        full((C, C)),
    ]
    out_specs = [
        pl.BlockSpec((1, C, H), lambda b, t: (b, t, 0)),
        pl.BlockSpec((1, C, H, H), lambda b, t: (b, t, 0, 0)),
    ]
    y, s4 = pl.pallas_call(
        _fwp_kernel,
        grid=(B, T // C),
        in_specs=in_specs,
        out_specs=out_specs,
        out_shape=[
            jax.ShapeDtypeStruct((B, T, H), jnp.float32),
            jax.ShapeDtypeStruct((B, T, H, H), jnp.float32),
        ],
        scratch_shapes=[
            pltpu.VMEM((H, H), jnp.float32),
        ],
        compiler_params=pltpu.CompilerParams(
            dimension_semantics=("parallel", "arbitrary"),
            vmem_limit_bytes=60 * 1024 * 1024,
        ),
    )(x, state_m, Wk, Wq, Wv, gamma.reshape(1, F),
      beta.reshape(1, F), W1, b1.reshape(1, H), W2, b2.reshape(1, H),
      Ws, bs.reshape(1, H), lexp, l32)
    return y, s4


# speedup vs baseline: 2.9065x; 2.9065x over previous
import jax
import jax.numpy as jnp
import numpy as np
from jax.experimental import pallas as pl
from jax.experimental.pallas import tpu as pltpu

EPS_LN = 1e-5
EPS_SUMNORM = 1e-5
B, T, F, H = 8, 1024, 128, 64
C = 256
SC = 128


def _fwp_kernel(xt_ref, st_rep_ref, st_mat_ref, wk_ref, wq_ref, wv_ref,
                g_ref, beta_ref, w1_ref, b1_ref, w2_ref, b2_ref, ws_ref,
                bs_ref, erep_ref, u2_ref, umask_ref,
                y_ref, s_ref, cs_ref, cm_ref):
    tc = pl.program_id(1)

    @pl.when(tc == 0)
    def _init():
        cs_ref[...] = st_rep_ref[0]
        cm_ref[...] = st_mat_ref[0]

    xt = xt_ref[0]
    mu = jnp.mean(xt, axis=0, keepdims=True)
    xc = xt - mu
    var = jnp.mean(xc * xc, axis=0, keepdims=True)
    tile2 = lambda a: jnp.tile(a, (1, C // SC))
    xn = (xc * jax.lax.rsqrt(var + EPS_LN)) * tile2(g_ref[...]) \
        + tile2(beta_ref[...])

    dg = lambda a, b: jax.lax.dot_general(
        a, b, (((1,), (0,)), ((), ())), preferred_element_type=jnp.float32)
    kt = jnp.maximum(dg(wk_ref[...], xn), 0.0)
    qt = jnp.maximum(dg(wq_ref[...], xn), 0.0)
    vt = dg(wv_ref[...], xn)
    kt = kt / (EPS_SUMNORM + jnp.sum(kt, axis=0, keepdims=True))
    qt = qt / (EPS_SUMNORM + jnp.sum(qt, axis=0, keepdims=True))

    vmat = jax.lax.dot_general(
        erep_ref[...], vt.astype(jnp.bfloat16), (((1,), (0,)), ((), ())),
        preferred_element_type=jnp.float32)
    kmat = jnp.tile(kt, (H, 1))
    kv16 = (vmat * kmat).astype(jnp.bfloat16)

    cs = cs_ref[...]
    u2 = u2_ref[...]
    sblks = []
    for p in range(C // SC):
        outp = jax.lax.dot_general(
            kv16[:, p * SC:(p + 1) * SC], u2, (((1,), (0,)), ((), ())),
            preferred_element_type=jnp.float32)
        sblks.append(outp[:, :SC] + cs)
        cs = cs + outp[:, SC:]
    cs_ref[...] = cs
    s_ref[0] = jnp.concatenate(sblks, axis=1).reshape(H, H, C)

    cm = cm_ref[...]
    at = jax.lax.dot_general(
        kt, qt, (((0,), (0,)), ((), ())),
        preferred_element_type=jnp.float32) * umask_ref[...]
    yt = dg(vt, at) + dg(cm, qt)

    tileb = lambda a: jnp.tile(a, (1, C // SC))
    h = jnp.maximum(dg(w1_ref[...], yt) + tileb(b1_ref[...]), 0.0)
    h = jnp.maximum(dg(w2_ref[...], h) + tileb(b2_ref[...]), 0.0)
    y_ref[0] = h + dg(ws_ref[...], xn) + tileb(bs_ref[...])

    cm_ref[...] = cm + jax.lax.dot_general(
        vt, kt, (((1,), (1,)), ((), ())), preferred_element_type=jnp.float32)


@jax.jit
def kernel(x, state, Wk, Wq, Wv, gamma, beta, W1, b1, W2, b2, Ws, bs):
    r = np.arange(H * H)
    erep = np.zeros((H * H, H), np.float32)
    erep[r, r // H] = 1.0
    erep = jnp.asarray(erep, dtype=jnp.bfloat16)
    s_i = np.arange(SC)
    u2 = np.concatenate(
        [(s_i[:, None] <= s_i[None, :]).astype(np.float32),
         np.ones((SC, SC), np.float32)], axis=1)
    u2 = jnp.asarray(u2, dtype=jnp.bfloat16)
    t_i = np.arange(C)
    umask = jnp.asarray((t_i[:, None] <= t_i[None, :]).astype(np.float32))

    xt = jnp.swapaxes(x, 1, 2)
    st_rep = jnp.broadcast_to(
        state.reshape(B, H * H, 1), (B, H * H, SC))
    st_mat = state.reshape(B, H, H)
    colb = lambda v, n: jnp.broadcast_to(v.reshape(n, 1), (n, SC))

    full = lambda shp: pl.BlockSpec(shp, lambda b, t: (0,) * len(shp))
    in_specs = [
        pl.BlockSpec((1, F, C), lambda b, t: (b, 0, t)),
        pl.BlockSpec((1, H * H, SC), lambda b, t: (b, 0, 0)),
        pl.BlockSpec((1, H, H), lambda b, t: (b, 0, 0)),
        full((H, F)), full((H, F)), full((H, F)),
        full((F, SC)), full((F, SC)),
        full((H, H)), full((H, SC)),
        full((H, H)), full((H, SC)),
        full((H, F)), full((H, SC)),
        full((H * H, H)),
        full((SC, 2 * SC)),
        full((C, C)),
    ]
    out_specs = [
        pl.BlockSpec((1, H, C), lambda b, t: (b, 0, t)),
        pl.BlockSpec((1, H, H, C), lambda b, t: (b, 0, 0, t)),
    ]
    yt, st = pl.pallas_call(
        _fwp_kernel,
        grid=(B, T // C),
        in_specs=in_specs,
        out_specs=out_specs,
        out_shape=[
            jax.ShapeDtypeStruct((B, H, T), jnp.float32),
            jax.ShapeDtypeStruct((B, H, H, T), jnp.float32),
        ],
        scratch_shapes=[
            pltpu.VMEM((H * H, SC), jnp.float32),
            pltpu.VMEM((H, H), jnp.float32),
        ],
        compiler_params=pltpu.CompilerParams(
            dimension_semantics=("parallel", "arbitrary"),
            vmem_limit_bytes=60 * 1024 * 1024,
        ),
    )(xt, st_rep, st_mat, Wk, Wq, Wv, colb(gamma, F), colb(beta, F),
      W1, colb(b1, H), W2, colb(b2, H), Ws, colb(bs, H), erep, u2, umask)
    return jnp.transpose(yt, (0, 2, 1)), jnp.transpose(st, (0, 3, 1, 2))


# speedup vs baseline: 3.1225x; 1.0743x over previous
import jax
import jax.numpy as jnp
import numpy as np
from jax.experimental import pallas as pl
from jax.experimental.pallas import tpu as pltpu

EPS_LN = 1e-5
EPS_SUMNORM = 1e-5
B, T, F, H = 8, 1024, 128, 64
C = 256
SC = 128
G = 2


def _fwp_kernel(xt_ref, st_rep_ref, st_mat_ref, wk_ref, wq_ref, wv_ref,
                g_ref, beta_ref, w1_ref, b1_ref, w2_ref, b2_ref, ws_ref,
                bs_ref, erep_ref, u2_ref, umask_ref,
                y_ref, s_ref, cs_ref, cm_ref):
    tc = pl.program_id(1)

    @pl.when(tc == 0)
    def _init():
        cs_ref[...] = st_rep_ref[...]
        cm_ref[...] = st_mat_ref[...]

    def one_batch(g):
        xt = xt_ref[g]
        mu = jnp.mean(xt, axis=0, keepdims=True)
        xc = xt - mu
        var = jnp.mean(xc * xc, axis=0, keepdims=True)
        tile2 = lambda a: jnp.tile(a, (1, C // SC))
        xn = (xc * jax.lax.rsqrt(var + EPS_LN)) * tile2(g_ref[...]) \
            + tile2(beta_ref[...])

        dg = lambda a, b: jax.lax.dot_general(
            a, b, (((1,), (0,)), ((), ())),
            preferred_element_type=jnp.float32)
        kt = jnp.maximum(dg(wk_ref[...], xn), 0.0)
        qt = jnp.maximum(dg(wq_ref[...], xn), 0.0)
        vt = dg(wv_ref[...], xn)
        kt = kt / (EPS_SUMNORM + jnp.sum(kt, axis=0, keepdims=True))
        qt = qt / (EPS_SUMNORM + jnp.sum(qt, axis=0, keepdims=True))

        vmat = jax.lax.dot_general(
            erep_ref[...], vt.astype(jnp.bfloat16), (((1,), (0,)), ((), ())),
            preferred_element_type=jnp.float32)
        kmat = jnp.tile(kt.astype(jnp.bfloat16), (H, 1))
        kv16 = vmat.astype(jnp.bfloat16) * kmat

        cs = cs_ref[g]
        outp = jax.lax.dot_general(
            kv16, u2_ref[...], (((1,), (0,)), ((), ())),
            preferred_element_type=jnp.float32)
        cs_ref[g] = cs + outp[:, C:]
        s_ref[g] = (outp[:, :C] + jnp.tile(cs, (1, C // SC))).reshape(H, H, C)

        cm = cm_ref[g]
        at = jax.lax.dot_general(
            kt, qt, (((0,), (0,)), ((), ())),
            preferred_element_type=jnp.float32) * umask_ref[...]
        yt = dg(vt, at) + dg(cm, qt)

        h = jnp.maximum(dg(w1_ref[...], yt) + tile2(b1_ref[...]), 0.0)
        h = jnp.maximum(dg(w2_ref[...], h) + tile2(b2_ref[...]), 0.0)
        y_ref[g] = h + dg(ws_ref[...], xn) + tile2(bs_ref[...])

        cm_ref[g] = cm + jax.lax.dot_general(
            vt, kt, (((1,), (1,)), ((), ())),
            preferred_element_type=jnp.float32)

    for g in range(G):
        one_batch(g)


@jax.jit
def kernel(x, state, Wk, Wq, Wv, gamma, beta, W1, b1, W2, b2, Ws, bs):
    r = np.arange(H * H)
    erep = np.zeros((H * H, H), np.float32)
    erep[r, r // H] = 1.0
    erep = jnp.asarray(erep, dtype=jnp.bfloat16)
    s_i = np.arange(C)
    u2 = np.concatenate(
        [(s_i[:, None] <= s_i[None, :]).astype(np.float32),
         np.ones((C, SC), np.float32)], axis=1)
    u2 = jnp.asarray(u2, dtype=jnp.bfloat16)
    umask = jnp.asarray((s_i[:, None] <= s_i[None, :]).astype(np.float32))

    xt = jnp.swapaxes(x, 1, 2)
    st_rep = jnp.broadcast_to(
        state.reshape(B, H * H, 1), (B, H * H, SC))
    st_mat = state.reshape(B, H, H)
    colb = lambda v, n: jnp.broadcast_to(v.reshape(n, 1), (n, SC))

    full = lambda shp: pl.BlockSpec(shp, lambda b, t: (0,) * len(shp))
    in_specs = [
        pl.BlockSpec((G, F, C), lambda b, t: (b, 0, t)),
        pl.BlockSpec((G, H * H, SC), lambda b, t: (b, 0, 0)),
        pl.BlockSpec((G, H, H), lambda b, t: (b, 0, 0)),
        full((H, F)), full((H, F)), full((H, F)),
        full((F, SC)), full((F, SC)),
        full((H, H)), full((H, SC)),
        full((H, H)), full((H, SC)),
        full((H, F)), full((H, SC)),
        full((H * H, H)),
        full((C, C + SC)),
        full((C, C)),
    ]
    out_specs = [
        pl.BlockSpec((G, H, C), lambda b, t: (b, 0, t)),
        pl.BlockSpec((G, H, H, C), lambda b, t: (b, 0, 0, t)),
    ]
    yt, st = pl.pallas_call(
        _fwp_kernel,
        grid=(B // G, T // C),
        in_specs=in_specs,
        out_specs=out_specs,
        out_shape=[
            jax.ShapeDtypeStruct((B, H, T), jnp.float32),
            jax.ShapeDtypeStruct((B, H, H, T), jnp.float32),
        ],
        scratch_shapes=[
            pltpu.VMEM((G, H * H, SC), jnp.float32),
            pltpu.VMEM((G, H, H), jnp.float32),
        ],
        compiler_params=pltpu.CompilerParams(
            dimension_semantics=("arbitrary", "arbitrary"),
            vmem_limit_bytes=60 * 1024 * 1024,
        ),
    )(xt, st_rep, st_mat, Wk, Wq, Wv, colb(gamma, F), colb(beta, F),
      W1, colb(b1, H), W2, colb(b2, H), Ws, colb(bs, H), erep, u2, umask)
    return jnp.transpose(yt, (0, 2, 1)), jnp.transpose(st, (0, 3, 1, 2))


# speedup vs baseline: 3.5123x; 1.1249x over previous
import jax
import jax.numpy as jnp
import numpy as np
from jax.experimental import pallas as pl
from jax.experimental.pallas import tpu as pltpu

EPS_LN = 1e-5
EPS_SUMNORM = 1e-5
B, T, F, H = 8, 1024, 128, 64
C = 256
SC = 128
G = 2


def _fwp_kernel(x_ref, st_mat_ref, wk_ref, wq_ref, wv_ref, bigb_ref,
                w1_ref, w2_ref, ws_ref, erep_ref, jrep2_ref, ones2_ref,
                u2_ref, umask_ref, y_ref, s_ref, cs_ref, cm_ref):
    tc = pl.program_id(1)

    @pl.when(tc == 0)
    def _init():
        cm_ref[...] = st_mat_ref[...]
        for g in range(G):
            st = st_mat_ref[g]
            hi = st.astype(jnp.bfloat16)
            lo = (st - hi.astype(jnp.float32)).astype(jnp.bfloat16)
            m = jax.lax.dot_general(
                erep_ref[...], jnp.concatenate([hi, lo], axis=1),
                (((1,), (0,)), ((), ())),
                preferred_element_type=jnp.float32)
            sel = m.astype(jnp.bfloat16) * jrep2_ref[...]
            cs_ref[g] = jax.lax.dot_general(
                sel, ones2_ref[...], (((1,), (0,)), ((), ())),
                preferred_element_type=jnp.float32)

    bigb = bigb_ref[...]
    gam = bigb[0:F]
    bet = bigb[F:2 * F]
    b1c = bigb[2 * F:2 * F + H]
    b2c = bigb[2 * F + H:2 * F + 2 * H]
    bsc = bigb[2 * F + 2 * H:2 * F + 3 * H]
    tile2 = lambda a: jnp.tile(a, (1, C // SC))

    def one_batch(g):
        xt = jnp.swapaxes(x_ref[g], 0, 1)
        mu = jnp.mean(xt, axis=0, keepdims=True)
        xc = xt - mu
        var = jnp.mean(xc * xc, axis=0, keepdims=True)
        xn = (xc * jax.lax.rsqrt(var + EPS_LN)) * tile2(gam) \
            + tile2(bet)

        dg = lambda a, b: jax.lax.dot_general(
            a, b, (((1,), (0,)), ((), ())),
            preferred_element_type=jnp.float32)
        kt = jnp.maximum(dg(wk_ref[...], xn), 0.0)
        qt = jnp.maximum(dg(wq_ref[...], xn), 0.0)
        vt = dg(wv_ref[...], xn)
        kt = kt / (EPS_SUMNORM + jnp.sum(kt, axis=0, keepdims=True))
        qt = qt / (EPS_SUMNORM + jnp.sum(qt, axis=0, keepdims=True))

        vmat = jax.lax.dot_general(
            erep_ref[...], vt.astype(jnp.bfloat16), (((1,), (0,)), ((), ())),
            preferred_element_type=jnp.float32)
        kmat = jnp.tile(kt.astype(jnp.bfloat16), (H, 1))
        kv16 = vmat.astype(jnp.bfloat16) * kmat

        cs = cs_ref[g]
        outp = jax.lax.dot_general(
            kv16, u2_ref[...], (((1,), (0,)), ((), ())),
            preferred_element_type=jnp.float32)
        cs_ref[g] = cs + outp[:, C:]
        s_ref[g] = (outp[:, :C] + jnp.tile(cs, (1, C // SC))).reshape(H, H, C)

        cm = cm_ref[g]
        at = jax.lax.dot_general(
            kt, qt, (((0,), (0,)), ((), ())),
            preferred_element_type=jnp.float32) * umask_ref[...]
        yt = dg(vt, at) + dg(cm, qt)

        h = jnp.maximum(dg(w1_ref[...], yt) + tile2(b1c), 0.0)
        h = jnp.maximum(dg(w2_ref[...], h) + tile2(b2c), 0.0)
        y_ref[g] = h + dg(ws_ref[...], xn) + tile2(bsc)

        cm_ref[g] = cm + jax.lax.dot_general(
            vt, kt, (((1,), (1,)), ((), ())),
            preferred_element_type=jnp.float32)

    for g in range(G):
        one_batch(g)


@jax.jit
def kernel(x, state, Wk, Wq, Wv, gamma, beta, W1, b1, W2, b2, Ws, bs):
    r = np.arange(H * H)
    erep = np.zeros((H * H, H), np.float32)
    erep[r, r // H] = 1.0
    erep = jnp.asarray(erep, dtype=jnp.bfloat16)
    jrep2 = np.zeros((H * H, 2 * H), np.float32)
    jrep2[r, r % H] = 1.0
    jrep2[r, H + r % H] = 1.0
    jrep2 = jnp.asarray(jrep2, dtype=jnp.bfloat16)
    ones2 = jnp.ones((2 * H, SC), dtype=jnp.bfloat16)
    s_i = np.arange(C)
    u2 = np.concatenate(
        [(s_i[:, None] <= s_i[None, :]).astype(np.float32),
         np.ones((C, SC), np.float32)], axis=1)
    u2 = jnp.asarray(u2, dtype=jnp.bfloat16)
    umask = jnp.asarray((s_i[:, None] <= s_i[None, :]).astype(np.float32))

    st_mat = state.reshape(B, H, H)
    colb = lambda v, n: jnp.broadcast_to(v.reshape(n, 1), (n, SC))
    bigb = jnp.concatenate(
        [colb(gamma, F), colb(beta, F), colb(b1, H), colb(b2, H),
         colb(bs, H)], axis=0)

    full = lambda shp: pl.BlockSpec(shp, lambda b, t: (0,) * len(shp))
    in_specs = [
        pl.BlockSpec((G, C, F), lambda b, t: (b, t, 0)),
        pl.BlockSpec((G, H, H), lambda b, t: (b, 0, 0)),
        full((H, F)), full((H, F)), full((H, F)),
        full((2 * F + 3 * H, SC)),
        full((H, H)), full((H, H)), full((H, F)),
        full((H * H, H)),
        full((H * H, 2 * H)),
        full((2 * H, SC)),
        full((C, C + SC)),
        full((C, C)),
    ]
    out_specs = [
        pl.BlockSpec((G, H, C), lambda b, t: (b, 0, t)),
        pl.BlockSpec((G, H, H, C), lambda b, t: (b, 0, 0, t)),
    ]
    yt, st = pl.pallas_call(
        _fwp_kernel,
        grid=(B // G, T // C),
        in_specs=in_specs,
        out_specs=out_specs,
        out_shape=[
            jax.ShapeDtypeStruct((B, H, T), jnp.float32),
            jax.ShapeDtypeStruct((B, H, H, T), jnp.float32),
        ],
        scratch_shapes=[
            pltpu.VMEM((G, H * H, SC), jnp.float32),
            pltpu.VMEM((G, H, H), jnp.float32),
        ],
        compiler_params=pltpu.CompilerParams(
            dimension_semantics=("arbitrary", "arbitrary"),
            vmem_limit_bytes=60 * 1024 * 1024,
        ),
    )(x, st_mat, Wk, Wq, Wv, bigb, W1, W2, Ws, erep, jrep2, ones2, u2, umask)
    return jnp.transpose(yt, (0, 2, 1)), jnp.transpose(st, (0, 3, 1, 2))


# speedup vs baseline: 4.1051x; 1.1688x over previous
import jax
import jax.numpy as jnp
import numpy as np
from jax.experimental import pallas as pl
from jax.experimental.pallas import tpu as pltpu

EPS_LN = 1e-5
EPS_SUMNORM = 1e-5
B, T, F, H = 8, 1024, 128, 64
C = 256
SC = 128
G = 2


def _fwp_kernel(x_ref, st_mat_ref, wk_ref, wq_ref, wv_ref, bigb_ref,
                w1_ref, w2_ref, ws_ref, erep_ref, jrep2_ref, ones2_ref,
                u2_ref, umask_ref, y_ref, s_ref, cs_ref, cm_ref):
    tc = pl.program_id(1)

    @pl.when(tc == 0)
    def _init():
        cm_ref[...] = st_mat_ref[...]
        for g in range(G):
            st = st_mat_ref[g]
            hi = st.astype(jnp.bfloat16)
            lo = (st - hi.astype(jnp.float32)).astype(jnp.bfloat16)
            m = jax.lax.dot_general(
                erep_ref[...], jnp.concatenate([hi, lo], axis=1),
                (((1,), (0,)), ((), ())),
                preferred_element_type=jnp.float32)
            sel = m.astype(jnp.bfloat16) * jrep2_ref[...]
            cs_ref[g] = jax.lax.dot_general(
                sel, ones2_ref[...], (((1,), (0,)), ((), ())),
                preferred_element_type=jnp.float32)

    bigb = bigb_ref[...]
    gam = bigb[0:F]
    bet = bigb[F:2 * F]
    b1c = bigb[2 * F:2 * F + H]
    b2c = bigb[2 * F + H:2 * F + 2 * H]
    bsc = bigb[2 * F + 2 * H:2 * F + 3 * H]
    tile2 = lambda a: jnp.tile(a, (1, C // SC))

    def one_batch(g):
        xt = jnp.swapaxes(x_ref[g], 0, 1)
        mu = jnp.mean(xt, axis=0, keepdims=True)
        xc = xt - mu
        var = jnp.mean(xc * xc, axis=0, keepdims=True)
        xn = (xc * jax.lax.rsqrt(var + EPS_LN)) * tile2(gam) \
            + tile2(bet)

        dg = lambda a, b: jax.lax.dot_general(
            a, b, (((1,), (0,)), ((), ())),
            preferred_element_type=jnp.float32)
        kt = jnp.maximum(dg(wk_ref[...], xn), 0.0)
        qt = jnp.maximum(dg(wq_ref[...], xn), 0.0)
        vt = dg(wv_ref[...], xn)
        kt = kt / (EPS_SUMNORM + jnp.sum(kt, axis=0, keepdims=True))
        qt = qt / (EPS_SUMNORM + jnp.sum(qt, axis=0, keepdims=True))

        vmat = jnp.repeat(vt.astype(jnp.bfloat16), H, axis=0)
        kmat = jnp.tile(kt.astype(jnp.bfloat16), (H, 1))
        kv16 = vmat * kmat

        cs = cs_ref[g]
        outp = jax.lax.dot_general(
            kv16, u2_ref[...], (((1,), (0,)), ((), ())),
            preferred_element_type=jnp.float32)
        cs_ref[g] = cs + outp[:, C:]
        s_ref[g] = (outp[:, :C] + jnp.tile(cs, (1, C // SC))).reshape(H, H, C)

        cm = cm_ref[g]
        at = jax.lax.dot_general(
            kt, qt, (((0,), (0,)), ((), ())),
            preferred_element_type=jnp.float32) * umask_ref[...]
        yt = dg(vt, at) + dg(cm, qt)

        h = jnp.maximum(dg(w1_ref[...], yt) + tile2(b1c), 0.0)
        h = jnp.maximum(dg(w2_ref[...], h) + tile2(b2c), 0.0)
        y_ref[g] = h + dg(ws_ref[...], xn) + tile2(bsc)

        cm_ref[g] = cm + jax.lax.dot_general(
            vt, kt, (((1,), (1,)), ((), ())),
            preferred_element_type=jnp.float32)

    for g in range(G):
        one_batch(g)


@jax.jit
def kernel(x, state, Wk, Wq, Wv, gamma, beta, W1, b1, W2, b2, Ws, bs):
    r = np.arange(H * H)
    erep = np.zeros((H * H, H), np.float32)
    erep[r, r // H] = 1.0
    erep = jnp.asarray(erep, dtype=jnp.bfloat16)
    jrep2 = np.zeros((H * H, 2 * H), np.float32)
    jrep2[r, r % H] = 1.0
    jrep2[r, H + r % H] = 1.0
    jrep2 = jnp.asarray(jrep2, dtype=jnp.bfloat16)
    ones2 = jnp.ones((2 * H, SC), dtype=jnp.bfloat16)
    s_i = np.arange(C)
    u2 = np.concatenate(
        [(s_i[:, None] <= s_i[None, :]).astype(np.float32),
         np.ones((C, SC), np.float32)], axis=1)
    u2 = jnp.asarray(u2, dtype=jnp.bfloat16)
    umask = jnp.asarray((s_i[:, None] <= s_i[None, :]).astype(np.float32))

    st_mat = state.reshape(B, H, H)
    colb = lambda v, n: jnp.broadcast_to(v.reshape(n, 1), (n, SC))
    bigb = jnp.concatenate(
        [colb(gamma, F), colb(beta, F), colb(b1, H), colb(b2, H),
         colb(bs, H)], axis=0)

    full = lambda shp: pl.BlockSpec(shp, lambda b, t: (0,) * len(shp))
    in_specs = [
        pl.BlockSpec((G, C, F), lambda b, t: (b, t, 0)),
        pl.BlockSpec((G, H, H), lambda b, t: (b, 0, 0)),
        full((H, F)), full((H, F)), full((H, F)),
        full((2 * F + 3 * H, SC)),
        full((H, H)), full((H, H)), full((H, F)),
        full((H * H, H)),
        full((H * H, 2 * H)),
        full((2 * H, SC)),
        full((C, C + SC)),
        full((C, C)),
    ]
    out_specs = [
        pl.BlockSpec((G, H, C), lambda b, t: (b, 0, t)),
        pl.BlockSpec((G, H, H, C), lambda b, t: (b, 0, 0, t)),
    ]
    yt, st = pl.pallas_call(
        _fwp_kernel,
        grid=(B // G, T // C),
        in_specs=in_specs,
        out_specs=out_specs,
        out_shape=[
            jax.ShapeDtypeStruct((B, H, T), jnp.float32),
            jax.ShapeDtypeStruct((B, H, H, T), jnp.float32),
        ],
        scratch_shapes=[
            pltpu.VMEM((G, H * H, SC), jnp.float32),
            pltpu.VMEM((G, H, H), jnp.float32),
        ],
        compiler_params=pltpu.CompilerParams(
            dimension_semantics=("arbitrary", "arbitrary"),
            vmem_limit_bytes=60 * 1024 * 1024,
        ),
    )(x, st_mat, Wk, Wq, Wv, bigb, W1, W2, Ws, erep, jrep2, ones2, u2, umask)
    return jnp.transpose(yt, (0, 2, 1)), jnp.transpose(st, (0, 3, 1, 2))


# speedup vs baseline: 4.3551x; 1.0609x over previous
import jax
import jax.numpy as jnp
import numpy as np
from jax.experimental import pallas as pl
from jax.experimental.pallas import tpu as pltpu

EPS_LN = 1e-5
EPS_SUMNORM = 1e-5
B, T, F, H = 8, 1024, 128, 64
C = 256
SC = 128
G = 2


def _fwp_kernel(x_ref, st_mat_ref, wk_ref, wq_ref, wv_ref, bigb_ref,
                w1_ref, w2_ref, ws_ref, erep_ref, jrep2_ref, ones2_ref,
                u2_ref, umask_ref, y_ref, s_ref, cs_ref, cm_ref):
    tc = pl.program_id(1)

    @pl.when(tc == 0)
    def _init():
        cm_ref[...] = st_mat_ref[...]
        for g in range(G):
            st = st_mat_ref[g]
            hi = st.astype(jnp.bfloat16)
            lo = (st - hi.astype(jnp.float32)).astype(jnp.bfloat16)
            m = jax.lax.dot_general(
                erep_ref[...], jnp.concatenate([hi, lo], axis=1),
                (((1,), (0,)), ((), ())),
                preferred_element_type=jnp.float32)
            sel = m.astype(jnp.bfloat16) * jrep2_ref[...]
            cs_ref[g] = jax.lax.dot_general(
                sel, ones2_ref[...], (((1,), (0,)), ((), ())),
                preferred_element_type=jnp.float32)

    bigb = bigb_ref[...]
    gam = bigb[0:F]
    bet = bigb[F:2 * F]
    b1c = bigb[2 * F:2 * F + H]
    b2c = bigb[2 * F + H:2 * F + 2 * H]
    bsc = bigb[2 * F + 2 * H:2 * F + 3 * H]
    tile2 = lambda a: jnp.tile(a, (1, C // SC))

    def one_batch(g):
        xt = jnp.swapaxes(x_ref[g], 0, 1)
        mu = jnp.mean(xt, axis=0, keepdims=True)
        xc = xt - mu
        var = jnp.mean(xc * xc, axis=0, keepdims=True)
        xn = (xc * jax.lax.rsqrt(var + EPS_LN)) * tile2(gam) \
            + tile2(bet)

        dg = lambda a, b: jax.lax.dot_general(
            a, b, (((1,), (0,)), ((), ())),
            preferred_element_type=jnp.float32)
        kt = jnp.maximum(dg(wk_ref[...], xn), 0.0)
        qt = jnp.maximum(dg(wq_ref[...], xn), 0.0)
        vt = dg(wv_ref[...], xn)
        kt = kt / (EPS_SUMNORM + jnp.sum(kt, axis=0, keepdims=True))
        qt = qt / (EPS_SUMNORM + jnp.sum(qt, axis=0, keepdims=True))

        vmat = jnp.repeat(vt.astype(jnp.bfloat16), H, axis=0)
        kmat = jnp.tile(kt.astype(jnp.bfloat16), (H, 1))
        kv16 = vmat * kmat

        cs = cs_ref[g]
        outp = jax.lax.dot_general(
            kv16, u2_ref[...], (((1,), (0,)), ((), ())),
            preferred_element_type=jnp.float32)
        cs_ref[g] = cs + jax.lax.broadcast_in_dim(
            outp[:, C - 1:C], (H * H, SC), (0, 1))
        s_ref[g] = (outp + jnp.tile(cs, (1, C // SC))).reshape(H, H, C)

        cm = cm_ref[g]
        at = jax.lax.dot_general(
            kt, qt, (((0,), (0,)), ((), ())),
            preferred_element_type=jnp.float32) * umask_ref[...]
        yt = dg(vt, at) + dg(cm, qt)

        h = jnp.maximum(dg(w1_ref[...], yt) + tile2(b1c), 0.0)
        h = jnp.maximum(dg(w2_ref[...], h) + tile2(b2c), 0.0)
        y_ref[g] = h + dg(ws_ref[...], xn) + tile2(bsc)

        cm_ref[g] = cm + jax.lax.dot_general(
            vt, kt, (((1,), (1,)), ((), ())),
            preferred_element_type=jnp.float32)

    for g in range(G):
        one_batch(g)


@jax.jit
def kernel(x, state, Wk, Wq, Wv, gamma, beta, W1, b1, W2, b2, Ws, bs):
    r = np.arange(H * H)
    erep = np.zeros((H * H, H), np.float32)
    erep[r, r // H] = 1.0
    erep = jnp.asarray(erep, dtype=jnp.bfloat16)
    jrep2 = np.zeros((H * H, 2 * H), np.float32)
    jrep2[r, r % H] = 1.0
    jrep2[r, H + r % H] = 1.0
    jrep2 = jnp.asarray(jrep2, dtype=jnp.bfloat16)
    ones2 = jnp.ones((2 * H, SC), dtype=jnp.bfloat16)
    s_i = np.arange(C)
    u2 = jnp.asarray(
        (s_i[:, None] <= s_i[None, :]).astype(np.float32),
        dtype=jnp.bfloat16)
    umask = jnp.asarray((s_i[:, None] <= s_i[None, :]).astype(np.float32))

    st_mat = state.reshape(B, H, H)
    colb = lambda v, n: jnp.broadcast_to(v.reshape(n, 1), (n, SC))
    bigb = jnp.concatenate(
        [colb(gamma, F), colb(beta, F), colb(b1, H), colb(b2, H),
         colb(bs, H)], axis=0)

    full = lambda shp: pl.BlockSpec(shp, lambda b, t: (0,) * len(shp))
    in_specs = [
        pl.BlockSpec((G, C, F), lambda b, t: (b, t, 0)),
        pl.BlockSpec((G, H, H), lambda b, t: (b, 0, 0)),
        full((H, F)), full((H, F)), full((H, F)),
        full((2 * F + 3 * H, SC)),
        full((H, H)), full((H, H)), full((H, F)),
        full((H * H, H)),
        full((H * H, 2 * H)),
        full((2 * H, SC)),
        full((C, C)),
        full((C, C)),
    ]
    out_specs = [
        pl.BlockSpec((G, H, C), lambda b, t: (b, 0, t)),
        pl.BlockSpec((G, H, H, C), lambda b, t: (b, 0, 0, t)),
    ]
    yt, st = pl.pallas_call(
        _fwp_kernel,
        grid=(B // G, T // C),
        in_specs=in_specs,
        out_specs=out_specs,
        out_shape=[
            jax.ShapeDtypeStruct((B, H, T), jnp.float32),
            jax.ShapeDtypeStruct((B, H, H, T), jnp.float32),
        ],
        scratch_shapes=[
            pltpu.VMEM((G, H * H, SC), jnp.float32),
            pltpu.VMEM((G, H, H), jnp.float32),
        ],
        compiler_params=pltpu.CompilerParams(
            dimension_semantics=("arbitrary", "arbitrary"),
            vmem_limit_bytes=60 * 1024 * 1024,
        ),
    )(x, st_mat, Wk, Wq, Wv, bigb, W1, W2, Ws, erep, jrep2, ones2, u2, umask)
    return jnp.transpose(yt, (0, 2, 1)), jnp.transpose(st, (0, 3, 1, 2))


# speedup vs baseline: 4.9694x; 1.1411x over previous
import jax
import jax.numpy as jnp
import numpy as np
from jax.experimental import pallas as pl
from jax.experimental.pallas import tpu as pltpu

EPS_LN = 1e-5
EPS_SUMNORM = 1e-5
B, T, F, H = 8, 1024, 128, 64
C = 256
SC = 128
G = 4


def _fwp_kernel(x_ref, st_mat_ref, wk_ref, wq_ref, wv_ref, bigb_ref,
                w1_ref, w2_ref, ws_ref, erep_ref, jrep2_ref, ones2_ref,
                u2_ref, umask_ref, y_ref, s_ref, cs_ref, cm_ref):
    tc = pl.program_id(1)

    @pl.when(tc == 0)
    def _init():
        cm_ref[...] = st_mat_ref[...]
        for g in range(G):
            st = st_mat_ref[g]
            hi = st.astype(jnp.bfloat16)
            lo = (st - hi.astype(jnp.float32)).astype(jnp.bfloat16)
            m = jax.lax.dot_general(
                erep_ref[...], jnp.concatenate([hi, lo], axis=1),
                (((1,), (0,)), ((), ())),
                preferred_element_type=jnp.float32)
            sel = m.astype(jnp.bfloat16) * jrep2_ref[...]
            cs_ref[g] = jax.lax.dot_general(
                sel, ones2_ref[...], (((1,), (0,)), ((), ())),
                preferred_element_type=jnp.float32)

    bigb = bigb_ref[...]
    gam = bigb[0:F]
    bet = bigb[F:2 * F]
    b1c = bigb[2 * F:2 * F + H]
    b2c = bigb[2 * F + H:2 * F + 2 * H]
    bsc = bigb[2 * F + 2 * H:2 * F + 3 * H]
    tile2 = lambda a: jnp.tile(a, (1, C // SC))

    def one_batch(g):
        xt = jnp.swapaxes(x_ref[g], 0, 1)
        mu = jnp.mean(xt, axis=0, keepdims=True)
        xc = xt - mu
        var = jnp.mean(xc * xc, axis=0, keepdims=True)
        xn = (xc * jax.lax.rsqrt(var + EPS_LN)) * tile2(gam) \
            + tile2(bet)

        dg = lambda a, b: jax.lax.dot_general(
            a, b, (((1,), (0,)), ((), ())),
            preferred_element_type=jnp.float32)
        kt = jnp.maximum(dg(wk_ref[...], xn), 0.0)
        qt = jnp.maximum(dg(wq_ref[...], xn), 0.0)
        vt = dg(wv_ref[...], xn)
        kt = kt / (EPS_SUMNORM + jnp.sum(kt, axis=0, keepdims=True))
        qt = qt / (EPS_SUMNORM + jnp.sum(qt, axis=0, keepdims=True))

        vmat = jnp.repeat(vt.astype(jnp.bfloat16), H, axis=0)
        kmat = jnp.tile(kt.astype(jnp.bfloat16), (H, 1))
        kv16 = vmat * kmat

        cs = cs_ref[g]
        outp = jax.lax.dot_general(
            kv16, u2_ref[...], (((1,), (0,)), ((), ())),
            preferred_element_type=jnp.float32)
        cs_ref[g] = cs + jax.lax.broadcast_in_dim(
            outp[:, C - 1:C], (H * H, SC), (0, 1))
        s_ref[g] = (outp + jnp.tile(cs, (1, C // SC))).reshape(H, H, C)

        cm = cm_ref[g]
        at = jax.lax.dot_general(
            kt, qt, (((0,), (0,)), ((), ())),
            preferred_element_type=jnp.float32) * umask_ref[...]
        yt = dg(vt, at) + dg(cm, qt)

        h = jnp.maximum(dg(w1_ref[...], yt) + tile2(b1c), 0.0)
        h = jnp.maximum(dg(w2_ref[...], h) + tile2(b2c), 0.0)
        y_ref[g] = h + dg(ws_ref[...], xn) + tile2(bsc)

        cm_ref[g] = cm + jax.lax.dot_general(
            vt, kt, (((1,), (1,)), ((), ())),
            preferred_element_type=jnp.float32)

    for g in range(G):
        one_batch(g)


@jax.jit
def kernel(x, state, Wk, Wq, Wv, gamma, beta, W1, b1, W2, b2, Ws, bs):
    r = np.arange(H * H)
    erep = np.zeros((H * H, H), np.float32)
    erep[r, r // H] = 1.0
    erep = jnp.asarray(erep, dtype=jnp.bfloat16)
    jrep2 = np.zeros((H * H, 2 * H), np.float32)
    jrep2[r, r % H] = 1.0
    jrep2[r, H + r % H] = 1.0
    jrep2 = jnp.asarray(jrep2, dtype=jnp.bfloat16)
    ones2 = jnp.ones((2 * H, SC), dtype=jnp.bfloat16)
    s_i = np.arange(C)
    u2 = jnp.asarray(
        (s_i[:, None] <= s_i[None, :]).astype(np.float32),
        dtype=jnp.bfloat16)
    umask = jnp.asarray((s_i[:, None] <= s_i[None, :]).astype(np.float32))

    st_mat = state.reshape(B, H, H)
    colb = lambda v, n: jnp.broadcast_to(v.reshape(n, 1), (n, SC))
    bigb = jnp.concatenate(
        [colb(gamma, F), colb(beta, F), colb(b1, H), colb(b2, H),
         colb(bs, H)], axis=0)

    full = lambda shp: pl.BlockSpec(shp, lambda b, t: (0,) * len(shp))
    in_specs = [
        pl.BlockSpec((G, C, F), lambda b, t: (b, t, 0)),
        pl.BlockSpec((G, H, H), lambda b, t: (b, 0, 0)),
        full((H, F)), full((H, F)), full((H, F)),
        full((2 * F + 3 * H, SC)),
        full((H, H)), full((H, H)), full((H, F)),
        full((H * H, H)),
        full((H * H, 2 * H)),
        full((2 * H, SC)),
        full((C, C)),
        full((C, C)),
    ]
    out_specs = [
        pl.BlockSpec((G, H, C), lambda b, t: (b, 0, t)),
        pl.BlockSpec((G, H, H, C), lambda b, t: (b, 0, 0, t)),
    ]
    yt, st = pl.pallas_call(
        _fwp_kernel,
        grid=(B // G, T // C),
        in_specs=in_specs,
        out_specs=out_specs,
        out_shape=[
            jax.ShapeDtypeStruct((B, H, T), jnp.float32),
            jax.ShapeDtypeStruct((B, H, H, T), jnp.float32),
        ],
        scratch_shapes=[
            pltpu.VMEM((G, H * H, SC), jnp.float32),
            pltpu.VMEM((G, H, H), jnp.float32),
        ],
        compiler_params=pltpu.CompilerParams(
            dimension_semantics=("arbitrary", "arbitrary"),
            vmem_limit_bytes=60 * 1024 * 1024,
        ),
    )(x, st_mat, Wk, Wq, Wv, bigb, W1, W2, Ws, erep, jrep2, ones2, u2, umask)
    return jnp.transpose(yt, (0, 2, 1)), jnp.transpose(st, (0, 3, 1, 2))
